# baseline (speedup 1.0000x reference)
import functools

import jax
import jax.numpy as jnp
from jax import lax
from jax.experimental import pallas as pl
from jax.experimental.pallas import tpu as pltpu
from jax.experimental.pallas import tpu_sc as plsc

_NC = 2
_NS = 16
_NW = _NC * _NS
_CHUNK = 128


def _proj_body(af_ref, bf_ref, waT_ref, wbT_ref, watT_ref, bn_ref, ba_ref,
               ap_ref, bp_ref, apre_ref, ast_ref):
    i = pl.program_id(0)
    af = af_ref[...]
    ap_ref[...] = jnp.dot(af, waT_ref[...], precision=lax.Precision.HIGHEST)
    bp_ref[...] = (jnp.dot(bf_ref[...], wbT_ref[...],
                           precision=lax.Precision.HIGHEST)
                   + bn_ref[0, :][None, :])
    apre = (jnp.dot(af, watT_ref[...], precision=lax.Precision.HIGHEST)
            + ba_ref[0, :][None, :])
    apre_ref[...] = apre

    @pl.when(i == 0)
    def _():
        ast_ref[...] = jnp.zeros_like(ast_ref)

    s = jnp.sum(apre, axis=0)
    q = jnp.sum(apre * apre, axis=0)
    pad = jnp.zeros((6, s.shape[0]), jnp.float32)
    ast_ref[...] += jnp.concatenate([s[None], q[None], pad], axis=0)


def _sc_gather_body(ap_hbm, bp_hbm, ia_hbm, ib_hbm, out_hbm, st_hbm,
                    idxa, idxb, ba0, bb0, bo0, ba1, bb1, bo1, stv,
                    sga0, sgb0, sga1, sgb1, sw0, sw1):
    wid = lax.axis_index("s") * _NC + lax.axis_index("c")
    rows_total = ia_hbm.shape[0]
    rpw = rows_total // _NW
    nchunk = rpw // _CHUNK
    base_w = wid * rpw

    pltpu.sync_copy(ia_hbm.at[pl.ds(base_w, rpw)], idxa)
    pltpu.sync_copy(ib_hbm.at[pl.ds(base_w, rpw)], idxb)

    bufsets = ((ba0, bb0, bo0, sga0, sgb0, sw0),
               (ba1, bb1, bo1, sga1, sgb1, sw1))

    def start_gather(g, ba, bb, sga, sgb):
        off = g * _CHUNK
        pltpu.async_copy(ap_hbm.at[idxa.at[pl.ds(off, _CHUNK)]], ba, sga)
        pltpu.async_copy(bp_hbm.at[idxb.at[pl.ds(off, _CHUNK)]], bb, sgb)

    def wait_gather(g, ba, bb, sga, sgb):
        off = g * _CHUNK
        pltpu.make_async_copy(ap_hbm.at[idxa.at[pl.ds(off, _CHUNK)]],
                              ba, sga).wait()
        pltpu.make_async_copy(bp_hbm.at[idxb.at[pl.ds(off, _CHUNK)]],
                              bb, sgb).wait()

    def out_slice(g):
        return out_hbm.at[pl.ds(base_w + g * _CHUNK, _CHUNK)]

    start_gather(0, ba0, bb0, sga0, sgb0)
    start_gather(1, ba1, bb1, sga1, sgb1)

    zero = jnp.zeros((16,), jnp.float32)
    acc0 = (zero,) * 8 + (zero,) * 8

    def super_chunk(h, acc):
        for p, (ba, bb, bo, sga, sgb, sw) in enumerate(bufsets):
            g = 2 * h + p
            wait_gather(g, ba, bb, sga, sgb)

            @pl.when(g >= 2)
            def _():
                pltpu.make_async_copy(bo, out_slice(g - 2), sw).wait()

            def rows2(r, acc_in):
                acc_out = acc_in
                for rr in range(2):
                    row = 2 * r + rr
                    ys = []
                    for j in range(8):
                        sl = pl.ds(j * 16, 16)
                        ys.append(ba[row, sl] + bb[row, sl])
                    for j in range(4):
                        z = plsc.pack(ys[2 * j], ys[2 * j + 1],
                                      format=plsc.PackFormat.INTERLEAVED)
                        bo[row, pl.ds(j * 16, 16)] = plsc.bitcast(z, jnp.int32)
                    acc_out = (tuple(acc_out[j] + ys[j] for j in range(8))
                               + tuple(acc_out[8 + j] + ys[j] * ys[j]
                                       for j in range(8)))
                return acc_out

            acc = lax.fori_loop(0, _CHUNK // 2, rows2, acc)

            @pl.when(g + 2 < nchunk)
            def _():
                start_gather(g + 2, ba, bb, sga, sgb)

            pltpu.async_copy(bo, out_slice(g), sw)
        return acc

    acc = lax.fori_loop(0, nchunk // 2, super_chunk, acc0)
    pltpu.make_async_copy(bo0, out_slice(nchunk - 2), sw0).wait()
    pltpu.make_async_copy(bo1, out_slice(nchunk - 1), sw1).wait()
    for j in range(16):
        stv[pl.ds(j * 16, 16)] = acc[j]
    pltpu.sync_copy(stv, st_hbm.at[wid])


def _nei_norm_body(x_ref, sc_ref, sh_ref, o_ref):
    x = x_ref[...]
    n = x.shape[1]
    lo = lax.bitcast_convert_type(x << 16, jnp.float32)
    hi = lax.bitcast_convert_type(x & jnp.int32(-65536), jnp.float32)
    y1 = lo * sc_ref[0, :n][None, :] + sh_ref[0, :n][None, :]
    y2 = hi * sc_ref[0, n:][None, :] + sh_ref[0, n:][None, :]
    o_ref[:, :n] = jnp.where(y1 >= 0, y1, 0.01 * y1)
    o_ref[:, n:] = jnp.where(y2 >= 0, y2, 0.01 * y2)


def _norm_body(x_ref, sc_ref, sh_ref, o_ref):
    x = x_ref[...].astype(jnp.float32)
    y = x * sc_ref[0, :][None, :] + sh_ref[0, :][None, :]
    o_ref[...] = jnp.where(y >= 0, y, 0.01 * y)


def kernel(atom_features, bond_features, atom_neighbor_list,
           bond_neighbor_list, W_atom, b_atom, gamma_atom, beta_atom,
           W_nei, b_nei, gamma_nei, beta_nei):
    B, A, AF = atom_features.shape
    NB, BF = bond_features.shape[1], bond_features.shape[2]
    K = atom_neighbor_list.shape[2]
    FP = W_atom.shape[0]
    NA = B * A
    NBR = B * NB
    ROWS = B * A * K

    af2 = atom_features.reshape(NA, AF)
    bf2 = bond_features.reshape(NBR, BF)
    boff = jnp.arange(B, dtype=jnp.int32)[:, None, None]
    ia = (atom_neighbor_list.astype(jnp.int32) + boff * A).reshape(ROWS)
    ib = (bond_neighbor_list.astype(jnp.int32) + boff * NB).reshape(ROWS)

    grp = jnp.arange(FP, dtype=jnp.int32) // 32
    i2 = jnp.arange(FP, dtype=jnp.int32) % 32
    pos = jnp.where(i2 < 16, 16 * grp + i2, FP // 2 + 16 * grp + (i2 - 16))
    inv = jnp.argsort(pos)

    waT = W_nei[:, :AF].T[:, pos]
    wbT = W_nei[:, AF:].T[:, pos]
    watT = W_atom.T
    bn8 = jnp.broadcast_to(b_nei[pos][None, :], (8, FP))
    ba8 = jnp.broadcast_to(b_atom[None, :], (8, FP))

    G1 = 16
    ca, cb = NA // G1, NBR // G1
    ap, bp, apre, astats = pl.pallas_call(
        _proj_body,
        grid=(G1,),
        in_specs=[
            pl.BlockSpec((ca, AF), lambda i: (i, 0)),
            pl.BlockSpec((cb, BF), lambda i: (i, 0)),
            pl.BlockSpec((AF, FP), lambda i: (0, 0)),
            pl.BlockSpec((BF, FP), lambda i: (0, 0)),
            pl.BlockSpec((AF, FP), lambda i: (0, 0)),
            pl.BlockSpec((8, FP), lambda i: (0, 0)),
            pl.BlockSpec((8, FP), lambda i: (0, 0)),
        ],
        out_specs=[
            pl.BlockSpec((ca, FP), lambda i: (i, 0)),
            pl.BlockSpec((cb, FP), lambda i: (i, 0)),
            pl.BlockSpec((ca, FP), lambda i: (i, 0)),
            pl.BlockSpec((8, FP), lambda i: (0, 0)),
        ],
        out_shape=[
            jax.ShapeDtypeStruct((NA, FP), jnp.float32),
            jax.ShapeDtypeStruct((NBR, FP), jnp.float32),
            jax.ShapeDtypeStruct((NA, FP), jnp.float32),
            jax.ShapeDtypeStruct((8, FP), jnp.float32),
        ],
    )(af2, bf2, waT, wbT, watT, bn8, ba8)

    rpw = ROWS // _NW
    mesh = plsc.VectorSubcoreMesh(core_axis_name="c", subcore_axis_name="s")
    sc_call = functools.partial(
        pl.kernel,
        mesh=mesh,
        compiler_params=pltpu.CompilerParams(needs_layout_passes=False),
        out_type=[
            jax.ShapeDtypeStruct((ROWS, FP // 2), jnp.int32),
            jax.ShapeDtypeStruct((_NW, 2 * FP), jnp.float32),
        ],
        scratch_types=[
            pltpu.VMEM((rpw,), jnp.int32),
            pltpu.VMEM((rpw,), jnp.int32),
            pltpu.VMEM((_CHUNK, FP), jnp.float32),
            pltpu.VMEM((_CHUNK, FP), jnp.float32),
            pltpu.VMEM((_CHUNK, FP // 2), jnp.int32),
            pltpu.VMEM((_CHUNK, FP), jnp.float32),
            pltpu.VMEM((_CHUNK, FP), jnp.float32),
            pltpu.VMEM((_CHUNK, FP // 2), jnp.int32),
            pltpu.VMEM((2 * FP,), jnp.float32),
            pltpu.SemaphoreType.DMA,
            pltpu.SemaphoreType.DMA,
            pltpu.SemaphoreType.DMA,
            pltpu.SemaphoreType.DMA,
            pltpu.SemaphoreType.DMA,
            pltpu.SemaphoreType.DMA,
        ],
    )
    nei_pre, nstats = sc_call(_sc_gather_body)(ap, bp, ia, ib)

    eps = 1e-6
    s_a, q_a = astats[0], astats[1]
    mean_a = s_a / NA
    var_a = q_a / NA - mean_a * mean_a
    sc_a = gamma_atom * lax.rsqrt(var_a + eps)
    sh_a = beta_atom - mean_a * sc_a

    s_n = jnp.sum(nstats[:, :FP], axis=0)[inv]
    q_n = jnp.sum(nstats[:, FP:], axis=0)[inv]
    mean_n = s_n / ROWS
    var_n = q_n / ROWS - mean_n * mean_n
    sc_n = gamma_nei * lax.rsqrt(var_n + eps)
    sh_n = beta_nei - mean_n * sc_n

    sc_n8 = jnp.broadcast_to(sc_n[None, :], (8, FP))
    sh_n8 = jnp.broadcast_to(sh_n[None, :], (8, FP))
    sc_a8 = jnp.broadcast_to(sc_a[None, :], (8, FP))
    sh_a8 = jnp.broadcast_to(sh_a[None, :], (8, FP))

    G3 = 64
    cn = ROWS // G3
    nei_fp = pl.pallas_call(
        _nei_norm_body,
        grid=(G3,),
        in_specs=[
            pl.BlockSpec((cn, FP // 2), lambda i: (i, 0)),
            pl.BlockSpec((8, FP), lambda i: (0, 0)),
            pl.BlockSpec((8, FP), lambda i: (0, 0)),
        ],
        out_specs=pl.BlockSpec((cn, FP), lambda i: (i, 0)),
        out_shape=jax.ShapeDtypeStruct((ROWS, FP), jnp.float32),
    )(nei_pre, sc_n8, sh_n8)

    G4 = 4
    cn4 = NA // G4
    atom_fp = pl.pallas_call(
        _norm_body,
        grid=(G4,),
        in_specs=[
            pl.BlockSpec((cn4, FP), lambda i: (i, 0)),
            pl.BlockSpec((8, FP), lambda i: (0, 0)),
            pl.BlockSpec((8, FP), lambda i: (0, 0)),
        ],
        out_specs=pl.BlockSpec((cn4, FP), lambda i: (i, 0)),
        out_shape=jax.ShapeDtypeStruct((NA, FP), jnp.float32),
    )(apre, sc_a8, sh_a8)

    return (atom_fp.reshape(B, A, FP), nei_fp.reshape(B, A, K, FP))


# speedup vs baseline: 1.1673x; 1.1673x over previous
import functools

import jax
import jax.numpy as jnp
from jax import lax
from jax.experimental import pallas as pl
from jax.experimental.pallas import tpu as pltpu
from jax.experimental.pallas import tpu_sc as plsc

_NC = 2
_NS = 16
_NW = _NC * _NS
_CHUNK = 128


def _proj_body(af_ref, bf_ref, waT_ref, wbT_ref, watT_ref, bn_ref, ba_ref,
               ap_ref, bp_ref, apre_ref, ast_ref):
    i = pl.program_id(0)
    af = af_ref[...]
    ap_ref[...] = jnp.dot(af, waT_ref[...], precision=lax.Precision.HIGHEST)
    bp_ref[...] = (jnp.dot(bf_ref[...], wbT_ref[...],
                           precision=lax.Precision.HIGHEST)
                   + bn_ref[0, :][None, :])
    apre = (jnp.dot(af, watT_ref[...], precision=lax.Precision.HIGHEST)
            + ba_ref[0, :][None, :])
    apre_ref[...] = apre

    @pl.when(i == 0)
    def _():
        ast_ref[...] = jnp.zeros_like(ast_ref)

    s = jnp.sum(apre, axis=0)
    q = jnp.sum(apre * apre, axis=0)
    pad = jnp.zeros((6, s.shape[0]), jnp.float32)
    ast_ref[...] += jnp.concatenate([s[None], q[None], pad], axis=0)


def _sc_gather_body(ap_hbm, bp_hbm, ia_hbm, ib_hbm, out_hbm, st_hbm,
                    idxa, idxb, ba0, bb0, bo0, ba1, bb1, bo1, stv,
                    sga0, sgb0, sga1, sgb1, sw0, sw1):
    wid = lax.axis_index("s") * _NC + lax.axis_index("c")
    rows_total = ia_hbm.shape[0]
    rpw = rows_total // _NW
    nchunk = rpw // _CHUNK
    base_w = wid * rpw

    pltpu.sync_copy(ia_hbm.at[pl.ds(base_w, rpw)], idxa)
    pltpu.sync_copy(ib_hbm.at[pl.ds(base_w, rpw)], idxb)

    bufsets = ((ba0, bb0, bo0, sga0, sgb0, sw0),
               (ba1, bb1, bo1, sga1, sgb1, sw1))

    def start_gather(g, ba, bb, sga, sgb):
        off = g * _CHUNK
        pltpu.async_copy(ap_hbm.at[idxa.at[pl.ds(off, _CHUNK)]], ba, sga)
        pltpu.async_copy(bp_hbm.at[idxb.at[pl.ds(off, _CHUNK)]], bb, sgb)

    def wait_gather(g, ba, bb, sga, sgb):
        off = g * _CHUNK
        pltpu.make_async_copy(ap_hbm.at[idxa.at[pl.ds(off, _CHUNK)]],
                              ba, sga).wait()
        pltpu.make_async_copy(bp_hbm.at[idxb.at[pl.ds(off, _CHUNK)]],
                              bb, sgb).wait()

    def out_slice(g):
        return out_hbm.at[pl.ds(wid * (rpw // 2) + g * (_CHUNK // 2),
                                _CHUNK // 2)]

    start_gather(0, ba0, bb0, sga0, sgb0)
    start_gather(1, ba1, bb1, sga1, sgb1)

    zero = jnp.zeros((16,), jnp.float32)
    acc0 = (zero,) * 8 + (zero,) * 8

    def super_chunk(h, acc):
        for p, (ba, bb, bo, sga, sgb, sw) in enumerate(bufsets):
            g = 2 * h + p
            wait_gather(g, ba, bb, sga, sgb)

            @pl.when(g >= 2)
            def _():
                pltpu.make_async_copy(bo, out_slice(g - 2), sw).wait()

            def rows2(r, acc_in):
                acc_out = acc_in
                ya, yb = [], []
                for j in range(8):
                    sl = pl.ds(j * 16, 16)
                    ya.append(ba[r, sl] + bb[r, sl])
                    yb.append(ba[r + 64, sl] + bb[r + 64, sl])
                for j in range(8):
                    z = plsc.pack(ya[j], yb[j],
                                  format=plsc.PackFormat.INTERLEAVED)
                    bo[r, pl.ds(j * 16, 16)] = plsc.bitcast(z, jnp.int32)
                acc_out = (tuple(acc_out[j] + ya[j] + yb[j] for j in range(8))
                           + tuple(acc_out[8 + j] + ya[j] * ya[j]
                                   + yb[j] * yb[j] for j in range(8)))
                return acc_out

            acc = lax.fori_loop(0, _CHUNK // 2, rows2, acc)

            @pl.when(g + 2 < nchunk)
            def _():
                start_gather(g + 2, ba, bb, sga, sgb)

            pltpu.async_copy(bo, out_slice(g), sw)
        return acc

    acc = lax.fori_loop(0, nchunk // 2, super_chunk, acc0)
    pltpu.make_async_copy(bo0, out_slice(nchunk - 2), sw0).wait()
    pltpu.make_async_copy(bo1, out_slice(nchunk - 1), sw1).wait()
    for j in range(16):
        stv[pl.ds(j * 16, 16)] = acc[j]
    pltpu.sync_copy(stv, st_hbm.at[wid])


def _nei_norm_body(x_ref, sc_ref, sh_ref, o_ref):
    scl = sc_ref[0, :][None, :]
    sht = sh_ref[0, :][None, :]
    ngrp = x_ref.shape[0] // 64
    for q in range(ngrp):
        x = x_ref[pl.ds(q * 64, 64), :]
        lo = lax.bitcast_convert_type(x << 16, jnp.float32)
        hi = lax.bitcast_convert_type(x & jnp.int32(-65536), jnp.float32)
        y1 = lo * scl + sht
        y2 = hi * scl + sht
        o_ref[pl.ds(q * 128, 64), :] = jnp.where(y1 >= 0, y1, 0.01 * y1)
        o_ref[pl.ds(q * 128 + 64, 64), :] = jnp.where(y2 >= 0, y2, 0.01 * y2)


def _norm_body(x_ref, sc_ref, sh_ref, o_ref):
    x = x_ref[...].astype(jnp.float32)
    y = x * sc_ref[0, :][None, :] + sh_ref[0, :][None, :]
    o_ref[...] = jnp.where(y >= 0, y, 0.01 * y)


def kernel(atom_features, bond_features, atom_neighbor_list,
           bond_neighbor_list, W_atom, b_atom, gamma_atom, beta_atom,
           W_nei, b_nei, gamma_nei, beta_nei):
    B, A, AF = atom_features.shape
    NB, BF = bond_features.shape[1], bond_features.shape[2]
    K = atom_neighbor_list.shape[2]
    FP = W_atom.shape[0]
    NA = B * A
    NBR = B * NB
    ROWS = B * A * K

    af2 = atom_features.reshape(NA, AF)
    bf2 = bond_features.reshape(NBR, BF)
    boff = jnp.arange(B, dtype=jnp.int32)[:, None, None]
    ia = (atom_neighbor_list.astype(jnp.int32) + boff * A).reshape(ROWS)
    ib = (bond_neighbor_list.astype(jnp.int32) + boff * NB).reshape(ROWS)

    waT = W_nei[:, :AF].T
    wbT = W_nei[:, AF:].T
    watT = W_atom.T
    bn8 = jnp.broadcast_to(b_nei[None, :], (8, FP))
    ba8 = jnp.broadcast_to(b_atom[None, :], (8, FP))

    G1 = 16
    ca, cb = NA // G1, NBR // G1
    ap, bp, apre, astats = pl.pallas_call(
        _proj_body,
        grid=(G1,),
        in_specs=[
            pl.BlockSpec((ca, AF), lambda i: (i, 0)),
            pl.BlockSpec((cb, BF), lambda i: (i, 0)),
            pl.BlockSpec((AF, FP), lambda i: (0, 0)),
            pl.BlockSpec((BF, FP), lambda i: (0, 0)),
            pl.BlockSpec((AF, FP), lambda i: (0, 0)),
            pl.BlockSpec((8, FP), lambda i: (0, 0)),
            pl.BlockSpec((8, FP), lambda i: (0, 0)),
        ],
        out_specs=[
            pl.BlockSpec((ca, FP), lambda i: (i, 0)),
            pl.BlockSpec((cb, FP), lambda i: (i, 0)),
            pl.BlockSpec((ca, FP), lambda i: (i, 0)),
            pl.BlockSpec((8, FP), lambda i: (0, 0)),
        ],
        out_shape=[
            jax.ShapeDtypeStruct((NA, FP), jnp.float32),
            jax.ShapeDtypeStruct((NBR, FP), jnp.float32),
            jax.ShapeDtypeStruct((NA, FP), jnp.float32),
            jax.ShapeDtypeStruct((8, FP), jnp.float32),
        ],
    )(af2, bf2, waT, wbT, watT, bn8, ba8)

    rpw = ROWS // _NW
    mesh = plsc.VectorSubcoreMesh(core_axis_name="c", subcore_axis_name="s")
    sc_call = functools.partial(
        pl.kernel,
        mesh=mesh,
        compiler_params=pltpu.CompilerParams(needs_layout_passes=False),
        out_type=[
            jax.ShapeDtypeStruct((ROWS // 2, FP), jnp.int32),
            jax.ShapeDtypeStruct((_NW, 2 * FP), jnp.float32),
        ],
        scratch_types=[
            pltpu.VMEM((rpw,), jnp.int32),
            pltpu.VMEM((rpw,), jnp.int32),
            pltpu.VMEM((_CHUNK, FP), jnp.float32),
            pltpu.VMEM((_CHUNK, FP), jnp.float32),
            pltpu.VMEM((_CHUNK // 2, FP), jnp.int32),
            pltpu.VMEM((_CHUNK, FP), jnp.float32),
            pltpu.VMEM((_CHUNK, FP), jnp.float32),
            pltpu.VMEM((_CHUNK // 2, FP), jnp.int32),
            pltpu.VMEM((2 * FP,), jnp.float32),
            pltpu.SemaphoreType.DMA,
            pltpu.SemaphoreType.DMA,
            pltpu.SemaphoreType.DMA,
            pltpu.SemaphoreType.DMA,
            pltpu.SemaphoreType.DMA,
            pltpu.SemaphoreType.DMA,
        ],
    )
    nei_pre, nstats = sc_call(_sc_gather_body)(ap, bp, ia, ib)

    eps = 1e-6
    s_a, q_a = astats[0], astats[1]
    mean_a = s_a / NA
    var_a = q_a / NA - mean_a * mean_a
    sc_a = gamma_atom * lax.rsqrt(var_a + eps)
    sh_a = beta_atom - mean_a * sc_a

    s_n = jnp.sum(nstats[:, :FP], axis=0)
    q_n = jnp.sum(nstats[:, FP:], axis=0)
    mean_n = s_n / ROWS
    var_n = q_n / ROWS - mean_n * mean_n
    sc_n = gamma_nei * lax.rsqrt(var_n + eps)
    sh_n = beta_nei - mean_n * sc_n

    sc_n8 = jnp.broadcast_to(sc_n[None, :], (8, FP))
    sh_n8 = jnp.broadcast_to(sh_n[None, :], (8, FP))
    sc_a8 = jnp.broadcast_to(sc_a[None, :], (8, FP))
    sh_a8 = jnp.broadcast_to(sh_a[None, :], (8, FP))

    G3 = 64
    cn = ROWS // G3
    nei_fp = pl.pallas_call(
        _nei_norm_body,
        grid=(G3,),
        in_specs=[
            pl.BlockSpec((cn // 2, FP), lambda i: (i, 0)),
            pl.BlockSpec((8, FP), lambda i: (0, 0)),
            pl.BlockSpec((8, FP), lambda i: (0, 0)),
        ],
        out_specs=pl.BlockSpec((cn, FP), lambda i: (i, 0)),
        out_shape=jax.ShapeDtypeStruct((ROWS, FP), jnp.float32),
    )(nei_pre, sc_n8, sh_n8)

    G4 = 4
    cn4 = NA // G4
    atom_fp = pl.pallas_call(
        _norm_body,
        grid=(G4,),
        in_specs=[
            pl.BlockSpec((cn4, FP), lambda i: (i, 0)),
            pl.BlockSpec((8, FP), lambda i: (0, 0)),
            pl.BlockSpec((8, FP), lambda i: (0, 0)),
        ],
        out_specs=pl.BlockSpec((cn4, FP), lambda i: (i, 0)),
        out_shape=jax.ShapeDtypeStruct((NA, FP), jnp.float32),
    )(apre, sc_a8, sh_a8)

    return (atom_fp.reshape(B, A, FP), nei_fp.reshape(B, A, K, FP))


# speedup vs baseline: 1.1714x; 1.0036x over previous
import functools

import jax
import jax.numpy as jnp
from jax import lax
from jax.experimental import pallas as pl
from jax.experimental.pallas import tpu as pltpu
from jax.experimental.pallas import tpu_sc as plsc

_NC = 2
_NS = 16
_NW = _NC * _NS
_CHUNK = 128


def _proj_body(af_ref, bf_ref, anl_ref, bnl_ref, waT_ref, wbT_ref, watT_ref,
               bn_ref, ba_ref, ap_ref, bp_ref, apre_ref, ast_ref, ia_ref,
               ib_ref, *, A, NB, rows_per_b):
    i = pl.program_id(0)
    af = af_ref[...]
    ap_ref[...] = jnp.dot(af, waT_ref[...], precision=lax.Precision.HIGHEST)
    bp_ref[...] = (jnp.dot(bf_ref[...], wbT_ref[...],
                           precision=lax.Precision.HIGHEST)
                   + bn_ref[0, :][None, :])
    apre = (jnp.dot(af, watT_ref[...], precision=lax.Precision.HIGHEST)
            + ba_ref[0, :][None, :])
    apre_ref[...] = apre

    nrow = anl_ref.shape[0]
    r = lax.broadcasted_iota(jnp.int32, (nrow, anl_ref.shape[1]), 0)
    b = i * (nrow // rows_per_b) + r // rows_per_b
    ia_ref[...] = anl_ref[...] + b * A
    ib_ref[...] = bnl_ref[...] + b * NB

    @pl.when(i == 0)
    def _():
        ast_ref[...] = jnp.zeros_like(ast_ref)

    s = jnp.sum(apre, axis=0)
    q = jnp.sum(apre * apre, axis=0)
    pad = jnp.zeros((6, s.shape[0]), jnp.float32)
    ast_ref[...] += jnp.concatenate([s[None], q[None], pad], axis=0)


def _sc_gather_body(ap_hbm, bp_hbm, ia_hbm, ib_hbm, out_hbm, st_hbm,
                    idxa, idxb, ba0, bb0, bo0, ba1, bb1, bo1, stv,
                    sga0, sgb0, sga1, sgb1, sw0, sw1):
    wid = lax.axis_index("s") * _NC + lax.axis_index("c")
    rows_total = ia_hbm.shape[0]
    rpw = rows_total // _NW
    nchunk = rpw // _CHUNK
    base_w = wid * rpw

    pltpu.sync_copy(ia_hbm.at[pl.ds(base_w, rpw)], idxa)
    pltpu.sync_copy(ib_hbm.at[pl.ds(base_w, rpw)], idxb)

    bufsets = ((ba0, bb0, bo0, sga0, sgb0, sw0),
               (ba1, bb1, bo1, sga1, sgb1, sw1))

    def start_gather(g, ba, bb, sga, sgb):
        off = g * _CHUNK
        pltpu.async_copy(ap_hbm.at[idxa.at[pl.ds(off, _CHUNK)]], ba, sga)
        pltpu.async_copy(bp_hbm.at[idxb.at[pl.ds(off, _CHUNK)]], bb, sgb)

    def wait_gather(g, ba, bb, sga, sgb):
        off = g * _CHUNK
        pltpu.make_async_copy(ap_hbm.at[idxa.at[pl.ds(off, _CHUNK)]],
                              ba, sga).wait()
        pltpu.make_async_copy(bp_hbm.at[idxb.at[pl.ds(off, _CHUNK)]],
                              bb, sgb).wait()

    def out_slice(g):
        return out_hbm.at[pl.ds(wid * (rpw // 2) + g * (_CHUNK // 2),
                                _CHUNK // 2)]

    start_gather(0, ba0, bb0, sga0, sgb0)
    start_gather(1, ba1, bb1, sga1, sgb1)

    zero = jnp.zeros((16,), jnp.float32)
    acc0 = (zero,) * 8 + (zero,) * 8

    def super_chunk(h, acc):
        for p, (ba, bb, bo, sga, sgb, sw) in enumerate(bufsets):
            g = 2 * h + p
            wait_gather(g, ba, bb, sga, sgb)

            @pl.when(g >= 2)
            def _():
                pltpu.make_async_copy(bo, out_slice(g - 2), sw).wait()

            def rows2(r, acc_in):
                acc_out = acc_in
                ya, yb = [], []
                for j in range(8):
                    sl = pl.ds(j * 16, 16)
                    ya.append(ba[r, sl] + bb[r, sl])
                    yb.append(ba[r + 64, sl] + bb[r + 64, sl])
                for j in range(8):
                    z = plsc.pack(ya[j], yb[j],
                                  format=plsc.PackFormat.INTERLEAVED)
                    bo[r, pl.ds(j * 16, 16)] = plsc.bitcast(z, jnp.int32)
                acc_out = (tuple(acc_out[j] + ya[j] + yb[j] for j in range(8))
                           + tuple(acc_out[8 + j] + ya[j] * ya[j]
                                   + yb[j] * yb[j] for j in range(8)))
                return acc_out

            acc = lax.fori_loop(0, _CHUNK // 2, rows2, acc)

            @pl.when(g + 2 < nchunk)
            def _():
                start_gather(g + 2, ba, bb, sga, sgb)

            pltpu.async_copy(bo, out_slice(g), sw)
        return acc

    acc = lax.fori_loop(0, nchunk // 2, super_chunk, acc0)
    pltpu.make_async_copy(bo0, out_slice(nchunk - 2), sw0).wait()
    pltpu.make_async_copy(bo1, out_slice(nchunk - 1), sw1).wait()
    for j in range(16):
        stv[pl.ds(j * 16, 16)] = acc[j]
    pltpu.sync_copy(stv, st_hbm.at[wid])


def _nei_norm_body(x_ref, st_ref, g_ref, b_ref, o_ref, *, nrows):
    st = st_ref[...]
    fp = g_ref.shape[1]
    s = jnp.sum(st[:, :fp], axis=0)
    q = jnp.sum(st[:, fp:], axis=0)
    mean = s / nrows
    var = q / nrows - mean * mean
    scl_v = g_ref[0, :] * lax.rsqrt(var + 1e-6)
    sht_v = b_ref[0, :] - mean * scl_v
    scl = scl_v[None, :]
    sht = sht_v[None, :]

    ngrp = x_ref.shape[0] // 64
    for q2 in range(ngrp):
        x = x_ref[pl.ds(q2 * 64, 64), :]
        lo = lax.bitcast_convert_type(x << 16, jnp.float32)
        hi = lax.bitcast_convert_type(x & jnp.int32(-65536), jnp.float32)
        y1 = lo * scl + sht
        y2 = hi * scl + sht
        o_ref[pl.ds(q2 * 128, 64), :] = jnp.where(y1 >= 0, y1, 0.01 * y1)
        o_ref[pl.ds(q2 * 128 + 64, 64), :] = jnp.where(y2 >= 0, y2, 0.01 * y2)


def _atom_norm_body(x_ref, st_ref, g_ref, b_ref, o_ref, *, nrows):
    st = st_ref[...]
    mean = st[0, :] / nrows
    var = st[1, :] / nrows - mean * mean
    scl = g_ref[0, :] * lax.rsqrt(var + 1e-6)
    sht = b_ref[0, :] - mean * scl
    y = x_ref[...] * scl[None, :] + sht[None, :]
    o_ref[...] = jnp.where(y >= 0, y, 0.01 * y)


def kernel(atom_features, bond_features, atom_neighbor_list,
           bond_neighbor_list, W_atom, b_atom, gamma_atom, beta_atom,
           W_nei, b_nei, gamma_nei, beta_nei):
    B, A, AF = atom_features.shape
    NB, BF = bond_features.shape[1], bond_features.shape[2]
    K = atom_neighbor_list.shape[2]
    FP = W_atom.shape[0]
    NA = B * A
    NBR = B * NB
    ROWS = B * A * K

    af2 = atom_features.reshape(NA, AF)
    bf2 = bond_features.reshape(NBR, BF)
    anl2 = atom_neighbor_list.astype(jnp.int32).reshape(ROWS // 128, 128)
    bnl2 = bond_neighbor_list.astype(jnp.int32).reshape(ROWS // 128, 128)

    waT = W_nei[:, :AF].T
    wbT = W_nei[:, AF:].T
    watT = W_atom.T
    bn8 = jnp.broadcast_to(b_nei[None, :], (8, FP))
    ba8 = jnp.broadcast_to(b_atom[None, :], (8, FP))

    G1 = 16
    ca, cb = NA // G1, NBR // G1
    ci = (ROWS // 128) // G1
    rows_per_b = (A * K) // 128
    ap, bp, apre, astats, ia2, ib2 = pl.pallas_call(
        functools.partial(_proj_body, A=A, NB=NB, rows_per_b=rows_per_b),
        grid=(G1,),
        in_specs=[
            pl.BlockSpec((ca, AF), lambda i: (i, 0)),
            pl.BlockSpec((cb, BF), lambda i: (i, 0)),
            pl.BlockSpec((ci, 128), lambda i: (i, 0)),
            pl.BlockSpec((ci, 128), lambda i: (i, 0)),
            pl.BlockSpec((AF, FP), lambda i: (0, 0)),
            pl.BlockSpec((BF, FP), lambda i: (0, 0)),
            pl.BlockSpec((AF, FP), lambda i: (0, 0)),
            pl.BlockSpec((8, FP), lambda i: (0, 0)),
            pl.BlockSpec((8, FP), lambda i: (0, 0)),
        ],
        out_specs=[
            pl.BlockSpec((ca, FP), lambda i: (i, 0)),
            pl.BlockSpec((cb, FP), lambda i: (i, 0)),
            pl.BlockSpec((ca, FP), lambda i: (i, 0)),
            pl.BlockSpec((8, FP), lambda i: (0, 0)),
            pl.BlockSpec((ci, 128), lambda i: (i, 0)),
            pl.BlockSpec((ci, 128), lambda i: (i, 0)),
        ],
        out_shape=[
            jax.ShapeDtypeStruct((NA, FP), jnp.float32),
            jax.ShapeDtypeStruct((NBR, FP), jnp.float32),
            jax.ShapeDtypeStruct((NA, FP), jnp.float32),
            jax.ShapeDtypeStruct((8, FP), jnp.float32),
            jax.ShapeDtypeStruct((ROWS // 128, 128), jnp.int32),
            jax.ShapeDtypeStruct((ROWS // 128, 128), jnp.int32),
        ],
    )(af2, bf2, anl2, bnl2, waT, wbT, watT, bn8, ba8)
    ia = ia2.reshape(ROWS)
    ib = ib2.reshape(ROWS)

    rpw = ROWS // _NW
    mesh = plsc.VectorSubcoreMesh(core_axis_name="c", subcore_axis_name="s")
    sc_call = functools.partial(
        pl.kernel,
        mesh=mesh,
        compiler_params=pltpu.CompilerParams(needs_layout_passes=False),
        out_type=[
            jax.ShapeDtypeStruct((ROWS // 2, FP), jnp.int32),
            jax.ShapeDtypeStruct((_NW, 2 * FP), jnp.float32),
        ],
        scratch_types=[
            pltpu.VMEM((rpw,), jnp.int32),
            pltpu.VMEM((rpw,), jnp.int32),
            pltpu.VMEM((_CHUNK, FP), jnp.float32),
            pltpu.VMEM((_CHUNK, FP), jnp.float32),
            pltpu.VMEM((_CHUNK // 2, FP), jnp.int32),
            pltpu.VMEM((_CHUNK, FP), jnp.float32),
            pltpu.VMEM((_CHUNK, FP), jnp.float32),
            pltpu.VMEM((_CHUNK // 2, FP), jnp.int32),
            pltpu.VMEM((2 * FP,), jnp.float32),
            pltpu.SemaphoreType.DMA,
            pltpu.SemaphoreType.DMA,
            pltpu.SemaphoreType.DMA,
            pltpu.SemaphoreType.DMA,
            pltpu.SemaphoreType.DMA,
            pltpu.SemaphoreType.DMA,
        ],
    )
    nei_pre, nstats = sc_call(_sc_gather_body)(ap, bp, ia, ib)

    gn8 = jnp.broadcast_to(gamma_nei[None, :], (8, FP))
    be8 = jnp.broadcast_to(beta_nei[None, :], (8, FP))
    ga8 = jnp.broadcast_to(gamma_atom[None, :], (8, FP))
    bt8 = jnp.broadcast_to(beta_atom[None, :], (8, FP))

    G3 = 64
    cn = ROWS // G3
    nei_fp = pl.pallas_call(
        functools.partial(_nei_norm_body, nrows=ROWS),
        grid=(G3,),
        in_specs=[
            pl.BlockSpec((cn // 2, FP), lambda i: (i, 0)),
            pl.BlockSpec((_NW, 2 * FP), lambda i: (0, 0)),
            pl.BlockSpec((8, FP), lambda i: (0, 0)),
            pl.BlockSpec((8, FP), lambda i: (0, 0)),
        ],
        out_specs=pl.BlockSpec((cn, FP), lambda i: (i, 0)),
        out_shape=jax.ShapeDtypeStruct((ROWS, FP), jnp.float32),
    )(nei_pre, nstats, gn8, be8)

    G4 = 4
    cn4 = NA // G4
    atom_fp = pl.pallas_call(
        functools.partial(_atom_norm_body, nrows=NA),
        grid=(G4,),
        in_specs=[
            pl.BlockSpec((cn4, FP), lambda i: (i, 0)),
            pl.BlockSpec((8, FP), lambda i: (0, 0)),
            pl.BlockSpec((8, FP), lambda i: (0, 0)),
            pl.BlockSpec((8, FP), lambda i: (0, 0)),
        ],
        out_specs=pl.BlockSpec((cn4, FP), lambda i: (i, 0)),
        out_shape=jax.ShapeDtypeStruct((NA, FP), jnp.float32),
    )(apre, astats, ga8, bt8)

    return (atom_fp.reshape(B, A, FP), nei_fp.reshape(B, A, K, FP))
